# baseline (speedup 1.0000x reference)
import functools

import jax
import jax.numpy as jnp
from jax import lax
from jax.experimental import pallas as pl
from jax.experimental.pallas import tpu as pltpu
from jax.experimental.pallas import tpu_sc as plsc

_B = 16384
_K = 16
_DIM = 64
_CH = 128
_APAD = 782
_RPK = 8
_PK = 8
_M = _B * _K // 128


def _make_sc_gather(nc, ns, nl):
    nw = nc * ns
    bpw = _B // nw
    nch = bpw // _CH
    opw = bpw * _K // 128
    mesh = plsc.VectorSubcoreMesh(core_axis_name="c", subcore_axis_name="s")

    @functools.partial(
        pl.kernel,
        out_type=(
            jax.ShapeDtypeStruct((_M, 128), jnp.float32),
            jax.ShapeDtypeStruct((_M, 128), jnp.float32),
            jax.ShapeDtypeStruct((_M, 128), jnp.float32),
        ),
        mesh=mesh,
        scratch_types=[
            pltpu.VMEM((bpw,), jnp.int32),
            pltpu.VMEM((bpw,), jnp.int32),
            pltpu.VMEM((bpw,), jnp.int32),
            pltpu.VMEM((bpw,), jnp.float32),
            pltpu.VMEM((_K,), jnp.float32),
            pltpu.VMEM((_CH, 128), jnp.float32),
            pltpu.VMEM((_CH, 128), jnp.float32),
            pltpu.VMEM((_CH, 128), jnp.float32),
            pltpu.VMEM((opw, 128), jnp.float32),
            pltpu.VMEM((opw, 128), jnp.float32),
            pltpu.VMEM((opw, 128), jnp.float32),
            pltpu.SemaphoreType.DMA,
        ],
        compiler_params=pltpu.CompilerParams(use_tc_tiling_on_sc=True,
                                             needs_layout_passes=False),
    )
    def gather(idx_hbm, tau_hbm, om_hbm, ap_hbm, A_hbm, P_hbm,
               ph_out, A_out, at_out,
               idx_v, idx3_v, idx7_v, tau_v, om_v,
               Ac, Pc, ac, ph_v, Av_v, at_v, sem):
        wid = lax.axis_index("s") * nc + lax.axis_index("c")
        base = wid * bpw
        pltpu.sync_copy(idx_hbm.at[pl.ds(base, bpw)], idx_v)
        pltpu.sync_copy(tau_hbm.at[pl.ds(base, bpw)], tau_v)
        pltpu.sync_copy(om_hbm, om_v)
        for t in range(bpw // nl):
            sl = pl.ds(t * nl, nl)
            v = idx_v[sl]
            idx3_v[sl] = v >> 3
            idx7_v[sl] = v >> 7
        omega = om_v[:]
        iota = lax.iota(jnp.int32, nl)
        for j in range(nch):
            i3 = idx3_v.at[pl.ds(j * _CH, _CH)]
            i7 = idx7_v.at[pl.ds(j * _CH, _CH)]
            ca = pltpu.async_copy(A_hbm.at[i3], Ac, sem)
            cp = pltpu.async_copy(P_hbm.at[i3], Pc, sem)
            cq = pltpu.async_copy(ap_hbm.at[i7], ac, sem)
            ca.wait()
            cp.wait()
            cq.wait()

            def row_block(t, _):
                for u in range(nl):
                    r = t * nl + u
                    b = j * _CH + r
                    bb = jnp.full((nl,), b, dtype=jnp.int32)
                    rr = jnp.full((nl,), r, dtype=jnp.int32)
                    rv = plsc.load_gather(idx_v, [bb])
                    tb = plsc.load_gather(tau_v, [bb])
                    colb = ((rv & 7) << 4) + iota
                    arow = plsc.load_gather(Ac, [rr, colb])
                    prow = plsc.load_gather(Pc, [rr, colb])
                    av = plsc.load_gather(ac, [rr, rv & 127])
                    orow = b >> 3
                    ocol = (b & 7) * _K
                    ph_v[orow, pl.ds(ocol, _K)] = tb * omega + prow
                    Av_v[orow, pl.ds(ocol, _K)] = arow
                    at_v[orow, pl.ds(ocol, _K)] = av * tb
                return _

            lax.fori_loop(0, _CH // nl, row_block, None)
        osl = pl.ds(wid * opw, opw)
        pltpu.sync_copy(ph_v, ph_out.at[osl])
        pltpu.sync_copy(Av_v, A_out.at[osl])
        pltpu.sync_copy(at_v, at_out.at[osl])

    return gather


def _dense_body(ph_ref, A_ref, at_ref, w_ref, b_ref, out_ref):
    zp = A_ref[:] * jnp.sin(ph_ref[:])
    zcat = jnp.concatenate([zp, at_ref[:]], axis=1)
    m = jnp.tanh(
        jnp.dot(zcat, w_ref[:], preferred_element_type=jnp.float32)
        + b_ref[:])
    cs = jnp.sum(m, axis=0, keepdims=True)
    mean = cs[:, 0:_DIM]
    for j in range(1, _PK):
        mean = mean + cs[:, j * _DIM:(j + 1) * _DIM]
    mean = mean * (1.0 / _B)
    mt = jnp.concatenate([mean] * _PK, axis=1)
    out_ref[:] = m - mt


def kernel(rel_id, tau, a_r, A_r, P_r, omega, W_proj, b_proj):
    info = plsc.get_sparse_core_info()
    gather = _make_sc_gather(info.num_cores, info.num_subcores,
                             info.num_lanes)
    a_pad = jnp.pad(a_r, (0, _APAD * 128 - a_r.shape[0])).reshape(_APAD, 128)
    A128 = A_r.reshape(-1, 128)
    P128 = P_r.reshape(-1, 128)
    ph, gA, at16 = gather(rel_id, tau, omega, a_pad, A128, P128)

    eye = jnp.eye(_PK, dtype=jnp.float32)
    wk = W_proj[:, 1:].T
    w0 = W_proj[:, 0]
    w_top = jnp.kron(eye, wk)
    w_bot = jnp.kron(eye, jnp.ones((_K, 1), jnp.float32) * (w0[None, :] / _K))
    w_ext = jnp.concatenate([w_top, w_bot], axis=0)
    b_tile = jnp.tile(b_proj, _PK)[None, :]

    m = pl.pallas_call(
        _dense_body,
        out_shape=jax.ShapeDtypeStruct((_M, _PK * _DIM), jnp.float32),
    )(ph, gA, at16, w_ext, b_tile)
    return m.reshape(_B, _DIM)


# speedup vs baseline: 1.0380x; 1.0380x over previous
import functools

import jax
import jax.numpy as jnp
from jax import lax
from jax.experimental import pallas as pl
from jax.experimental.pallas import tpu as pltpu
from jax.experimental.pallas import tpu_sc as plsc

_B = 16384
_K = 16
_DIM = 64
_CH = 128
_APAD = 782
_RPK = 8
_PK = 8
_M = _B * _K // 128


def _make_sc_gather(nc, ns, nl):
    nw = nc * ns
    bpw = _B // nw
    nch = bpw // _CH
    opw = bpw * _K // 128
    mesh = plsc.VectorSubcoreMesh(core_axis_name="c", subcore_axis_name="s")

    @functools.partial(
        pl.kernel,
        out_type=(
            jax.ShapeDtypeStruct((_M, 128), jnp.float32),
            jax.ShapeDtypeStruct((_M, 128), jnp.float32),
            jax.ShapeDtypeStruct((_M, 128), jnp.float32),
        ),
        mesh=mesh,
        scratch_types=[
            pltpu.VMEM((bpw,), jnp.int32),
            pltpu.VMEM((bpw,), jnp.int32),
            pltpu.VMEM((bpw,), jnp.float32),
            pltpu.VMEM((_K,), jnp.float32),
            pltpu.VMEM((_CH, _K), jnp.float32),
            pltpu.VMEM((_CH, _K), jnp.float32),
            pltpu.VMEM((_CH, 128), jnp.float32),
            pltpu.VMEM((opw, 128), jnp.float32),
            pltpu.VMEM((opw, 128), jnp.float32),
            pltpu.VMEM((opw, 128), jnp.float32),
            pltpu.SemaphoreType.DMA,
        ],
        compiler_params=pltpu.CompilerParams(use_tc_tiling_on_sc=False,
                                             needs_layout_passes=False),
    )
    def gather(idx_hbm, tau_hbm, om_hbm, ap_hbm, A_hbm, P_hbm,
               ph_out, A_out, at_out,
               idx_v, idx7_v, tau_v, om_v,
               Ac, Pc, ac, ph_v, Av_v, at_v, sem):
        wid = lax.axis_index("s") * nc + lax.axis_index("c")
        base = wid * bpw
        pltpu.sync_copy(idx_hbm.at[pl.ds(base, bpw)], idx_v)
        pltpu.sync_copy(tau_hbm.at[pl.ds(base, bpw)], tau_v)
        pltpu.sync_copy(om_hbm, om_v)
        for t in range(bpw // nl):
            sl = pl.ds(t * nl, nl)
            idx7_v[sl] = idx_v[sl] >> 7
        omega = om_v[:]
        iota = lax.iota(jnp.int32, nl)
        for j in range(nch):
            iv = idx_v.at[pl.ds(j * _CH, _CH)]
            i7 = idx7_v.at[pl.ds(j * _CH, _CH)]
            ca = pltpu.async_copy(A_hbm.at[iv], Ac, sem)
            cp = pltpu.async_copy(P_hbm.at[iv], Pc, sem)
            cq = pltpu.async_copy(ap_hbm.at[i7], ac, sem)
            ca.wait()
            cp.wait()
            cq.wait()

            def row_block(t, _):
                for u in range(nl):
                    r = t * nl + u
                    b = j * _CH + r
                    bb = jnp.full((nl,), b, dtype=jnp.int32)
                    rr = jnp.full((nl,), r, dtype=jnp.int32)
                    rv = plsc.load_gather(idx_v, [bb])
                    tb = plsc.load_gather(tau_v, [bb])
                    arow = plsc.load_gather(Ac, [rr, iota])
                    prow = plsc.load_gather(Pc, [rr, iota])
                    av = plsc.load_gather(ac, [rr, rv & 127])
                    orow = b >> 3
                    ocol = (b & 7) * _K
                    ph_v[orow, pl.ds(ocol, _K)] = tb * omega + prow
                    Av_v[orow, pl.ds(ocol, _K)] = arow
                    at_v[orow, pl.ds(ocol, _K)] = av * tb
                return _

            lax.fori_loop(0, _CH // nl, row_block, None)
        osl = pl.ds(wid * opw, opw)
        pltpu.sync_copy(ph_v, ph_out.at[osl])
        pltpu.sync_copy(Av_v, A_out.at[osl])
        pltpu.sync_copy(at_v, at_out.at[osl])

    return gather


def _dense_body(ph_ref, A_ref, at_ref, w_ref, b_ref, out_ref):
    zp = A_ref[:] * jnp.sin(ph_ref[:])
    zcat = jnp.concatenate([zp, at_ref[:]], axis=1)
    m = jnp.tanh(
        jnp.dot(zcat, w_ref[:], preferred_element_type=jnp.float32)
        + b_ref[:])
    cs = jnp.sum(m, axis=0, keepdims=True)
    mean = cs[:, 0:_DIM]
    for j in range(1, _PK):
        mean = mean + cs[:, j * _DIM:(j + 1) * _DIM]
    mean = mean * (1.0 / _B)
    mt = jnp.concatenate([mean] * _PK, axis=1)
    out_ref[:] = m - mt


def kernel(rel_id, tau, a_r, A_r, P_r, omega, W_proj, b_proj):
    info = plsc.get_sparse_core_info()
    gather = _make_sc_gather(info.num_cores, info.num_subcores,
                             info.num_lanes)
    a_pad = jnp.pad(a_r, (0, _APAD * 128 - a_r.shape[0])).reshape(_APAD, 128)
    ph, gA, at16 = gather(rel_id, tau, omega, a_pad, A_r, P_r)

    eye = jnp.eye(_PK, dtype=jnp.float32)
    wk = W_proj[:, 1:].T
    w0 = W_proj[:, 0]
    w_top = jnp.kron(eye, wk)
    w_bot = jnp.kron(eye, jnp.ones((_K, 1), jnp.float32) * (w0[None, :] / _K))
    w_ext = jnp.concatenate([w_top, w_bot], axis=0)
    b_tile = jnp.tile(b_proj, _PK)[None, :]

    m = pl.pallas_call(
        _dense_body,
        out_shape=jax.ShapeDtypeStruct((_M, _PK * _DIM), jnp.float32),
    )(ph, gA, at16, w_ext, b_tile)
    return m.reshape(_B, _DIM)
